# baseline (speedup 1.0000x reference)
import functools

import jax
import jax.numpy as jnp
from jax import lax
from jax.experimental import pallas as pl
from jax.experimental.pallas import tpu as pltpu
from jax.experimental.pallas import tpu_sc as plsc

B, L, D = 1024, 50, 768
N = B * L
EPS = 1e-12
NL = 16
NJ = D // NL

NC, NS = 2, 16
NW = NC * NS
SPW = B // NW
LP = 56


def _allsum(v):
    for sh in (8, 4, 2, 1):
        idx = lax.iota(jnp.int32, NL) ^ sh
        v = v + lax.gather(
            v, idx[:, None],
            lax.GatherDimensionNumbers(
                offset_dims=(), collapsed_slice_dims=(0,),
                start_index_map=(0,)),
            slice_sizes=(1,),
            mode=lax.GatherScatterMode.PROMISE_IN_BOUNDS)
    return v


def _ln_rows(rows_v, padd_v, g_v, b_v, i):
    x = []
    sum_v = jnp.zeros((NL,), jnp.float32)
    sq_v = jnp.zeros((NL,), jnp.float32)
    for j in range(NJ):
        v = rows_v[i, pl.ds(j * NL, NL)] + padd_v[pl.ds(i * D + j * NL, NL)]
        x.append(v)
        sum_v = sum_v + v
        sq_v = sq_v + v * v
    mu = _allsum(sum_v) * (1.0 / D)
    var = _allsum(sq_v) * (1.0 / D) - mu * mu
    xr = var + EPS
    seed = jnp.full((NL,), 0x5F3759DF, dtype=jnp.int32) - (
        lax.bitcast_convert_type(xr, jnp.int32) >> 1)
    y = lax.bitcast_convert_type(seed, jnp.float32)
    for _ in range(3):
        y = y * (1.5 - 0.5 * xr * y * y)
    for j in range(NJ):
        g = g_v[pl.ds(j * NL, NL)]
        bta = b_v[pl.ds(j * NL, NL)]
        rows_v[i, pl.ds(j * NL, NL)] = (x[j] - mu) * y * g + bta


def _sc_fused(ids3, table, padd, gamma, beta):
    mesh = plsc.VectorSubcoreMesh(core_axis_name="c", subcore_axis_name="s")

    @functools.partial(
        pl.kernel,
        mesh=mesh,
        out_type=jax.ShapeDtypeStruct((B, LP, D), jnp.float32),
        scratch_types=[
            pltpu.VMEM((SPW * LP,), jnp.int32),
            pltpu.VMEM((LP, D), jnp.float32),
            pltpu.VMEM((LP, D), jnp.float32),
            pltpu.VMEM((L * D,), jnp.float32),
            pltpu.VMEM((D,), jnp.float32),
            pltpu.VMEM((D,), jnp.float32),
            pltpu.SemaphoreType.DMA,
            pltpu.SemaphoreType.DMA,
            pltpu.SemaphoreType.DMA,
            pltpu.SemaphoreType.DMA,
        ],
    )
    def k(ids_hbm, table_hbm, padd_hbm, g_hbm, b_hbm, out_hbm,
          idx_v, rows_a, rows_b, padd_v, g_v, b_v, gs_a, gs_b, os_a, os_b):
        wid = lax.axis_index("s") * NC + lax.axis_index("c")
        seq0 = wid * SPW

        pltpu.sync_copy(ids_hbm.at[wid], idx_v)
        pltpu.sync_copy(padd_hbm, padd_v)
        pltpu.sync_copy(g_hbm, g_v)
        pltpu.sync_copy(b_hbm, b_v)

        rows = (rows_a, rows_b)
        gsem = (gs_a, gs_b)
        osem = (os_a, os_b)

        def gather_start(s, buf):
            pltpu.make_async_copy(
                table_hbm.at[idx_v.at[pl.ds(s * LP, LP)]], rows[buf],
                gsem[buf]).start()

        def gather_wait(buf):
            pltpu.make_async_copy(
                table_hbm.at[idx_v.at[pl.ds(0, LP)]], rows[buf],
                gsem[buf]).wait()

        def out_start(s, buf):
            pltpu.make_async_copy(
                rows[buf], out_hbm.at[seq0 + s], osem[buf]).start()

        def out_wait(buf):
            pltpu.make_async_copy(
                rows[buf], out_hbm.at[seq0], osem[buf]).wait()

        def compute(buf):
            def body(i, c):
                _ln_rows(rows[buf], padd_v, g_v, b_v, i)
                return c
            lax.fori_loop(0, L, body, 0)

        gather_start(0, 0)
        gather_start(1, 1)

        def phase(s, buf, issue_next):
            gather_wait(buf)
            compute(buf)
            out_start(s, buf)
            out_wait(buf)
            if issue_next:
                gather_start(s + 2, buf)

        def loop_body(ss, c):
            for buf in range(2):
                phase(ss * 2 + buf, buf, True)
            return c

        lax.fori_loop(0, SPW // 2 - 1, loop_body, 0)
        for buf in range(2):
            phase(SPW - 2 + buf, buf, False)

    return k(ids3, table, padd, gamma, beta)


def kernel(input_ids, word_emb, pos_emb, type_emb, ln_gamma, ln_beta):
    ids_p = jnp.pad(input_ids.astype(jnp.int32), ((0, 0), (0, LP - L)))
    ids3 = ids_p.reshape(NW, SPW * LP)
    padd = (pos_emb[:L] + type_emb[0][None, :]).reshape(-1)
    out_p = _sc_fused(ids3, word_emb, padd, ln_gamma, ln_beta)
    return out_p[:, :L, :]


# speedup vs baseline: 1.1527x; 1.1527x over previous
import functools

import jax
import jax.numpy as jnp
from jax import lax
from jax.experimental import pallas as pl
from jax.experimental.pallas import tpu as pltpu
from jax.experimental.pallas import tpu_sc as plsc

B, L, D = 1024, 50, 768
N = B * L
EPS = 1e-12
NL = 16
NJ = D // NL

NC, NS = 2, 16
NW = NC * NS
SPW = B // NW
LP = 56


def _allsum(v):
    for sh in (8, 4, 2, 1):
        idx = lax.iota(jnp.int32, NL) ^ sh
        v = v + lax.gather(
            v, idx[:, None],
            lax.GatherDimensionNumbers(
                offset_dims=(), collapsed_slice_dims=(0,),
                start_index_map=(0,)),
            slice_sizes=(1,),
            mode=lax.GatherScatterMode.PROMISE_IN_BOUNDS)
    return v


def _ln_rows(rows_v, padd_v, g_v, b_v, i):
    x = []
    sum_v = jnp.zeros((NL,), jnp.float32)
    sq_v = jnp.zeros((NL,), jnp.float32)
    for j in range(NJ):
        v = rows_v[i, pl.ds(j * NL, NL)] + padd_v[pl.ds(i * D + j * NL, NL)]
        x.append(v)
        sum_v = sum_v + v
        sq_v = sq_v + v * v
    mu = _allsum(sum_v) * (1.0 / D)
    var = _allsum(sq_v) * (1.0 / D) - mu * mu
    xr = var + EPS
    seed = jnp.full((NL,), 0x5F3759DF, dtype=jnp.int32) - (
        lax.bitcast_convert_type(xr, jnp.int32) >> 1)
    y = lax.bitcast_convert_type(seed, jnp.float32)
    for _ in range(3):
        y = y * (1.5 - 0.5 * xr * y * y)
    for j in range(NJ):
        g = g_v[pl.ds(j * NL, NL)]
        bta = b_v[pl.ds(j * NL, NL)]
        rows_v[i, pl.ds(j * NL, NL)] = (x[j] - mu) * y * g + bta


def _sc_fused(ids3, table, padd, gamma, beta):
    mesh = plsc.VectorSubcoreMesh(core_axis_name="c", subcore_axis_name="s")

    @functools.partial(
        pl.kernel,
        mesh=mesh,
        out_type=jax.ShapeDtypeStruct((B, LP, D), jnp.float32),
        scratch_types=[
            pltpu.VMEM((SPW * LP,), jnp.int32),
            pltpu.VMEM((LP, D), jnp.float32),
            pltpu.VMEM((LP, D), jnp.float32),
            pltpu.VMEM((L * D,), jnp.float32),
            pltpu.VMEM((D,), jnp.float32),
            pltpu.VMEM((D,), jnp.float32),
            pltpu.SemaphoreType.DMA,
            pltpu.SemaphoreType.DMA,
            pltpu.SemaphoreType.DMA,
            pltpu.SemaphoreType.DMA,
        ],
    )
    def k(ids_hbm, table_hbm, padd_hbm, g_hbm, b_hbm, out_hbm,
          idx_v, rows_a, rows_b, padd_v, g_v, b_v, gs_a, gs_b, os_a, os_b):
        wid = lax.axis_index("s") * NC + lax.axis_index("c")
        seq0 = wid * SPW

        pltpu.sync_copy(ids_hbm.at[wid], idx_v)
        pltpu.sync_copy(padd_hbm, padd_v)
        pltpu.sync_copy(g_hbm, g_v)
        pltpu.sync_copy(b_hbm, b_v)

        rows = (rows_a, rows_b)
        gsem = (gs_a, gs_b)
        osem = (os_a, os_b)

        def gather_start(s, buf):
            pltpu.make_async_copy(
                table_hbm.at[idx_v.at[pl.ds(s * LP, LP)]], rows[buf],
                gsem[buf]).start()

        def gather_wait(buf):
            pltpu.make_async_copy(
                table_hbm.at[idx_v.at[pl.ds(0, LP)]], rows[buf],
                gsem[buf]).wait()

        def out_start(s, buf):
            pltpu.make_async_copy(
                rows[buf], out_hbm.at[seq0 + s], osem[buf]).start()

        def out_wait(buf):
            pltpu.make_async_copy(
                rows[buf], out_hbm.at[seq0], osem[buf]).wait()

        def compute(buf):
            @plsc.parallel_loop(0, L, unroll=2)
            def _(i):
                _ln_rows(rows[buf], padd_v, g_v, b_v, i)

        gather_start(0, 0)
        gather_start(1, 1)

        def phase(s, buf, issue_next):
            gather_wait(buf)
            compute(buf)
            out_start(s, buf)
            out_wait(buf)
            if issue_next:
                gather_start(s + 2, buf)

        def loop_body(ss, c):
            for buf in range(2):
                phase(ss * 2 + buf, buf, True)
            return c

        lax.fori_loop(0, SPW // 2 - 1, loop_body, 0)
        for buf in range(2):
            phase(SPW - 2 + buf, buf, False)

    return k(ids3, table, padd, gamma, beta)


def kernel(input_ids, word_emb, pos_emb, type_emb, ln_gamma, ln_beta):
    ids_p = jnp.pad(input_ids.astype(jnp.int32), ((0, 0), (0, LP - L)))
    ids3 = ids_p.reshape(NW, SPW * LP)
    padd = (pos_emb[:L] + type_emb[0][None, :]).reshape(-1)
    out_p = _sc_fused(ids3, word_emb, padd, ln_gamma, ln_beta)
    return out_p[:, :L, :]
